# Initial kernel scaffold; baseline (speedup 1.0000x reference)
import jax
import jax.numpy as jnp
from jax.experimental import pallas as pl


def kernel(x, time, cond, pos, edge_index, p1W, p1b, p2W, p2b, inW1, inb1, inW2, inb2, posW1, posb1, posW2, posb2, attn, Wt1, bt1, Wt2, bt2, Wc1, bc1, Wc2, bc2, gnw, gnb, tnw, tnb, tcW, tcb, outW1, outb1, outW2, outb2):
    raise NotImplementedError("write your pallas kernel here")


# speedup vs baseline: 82.7892x; 82.7892x over previous
import functools

import jax
import jax.numpy as jnp
from jax import lax
from jax.experimental import pallas as pl
from jax.experimental.pallas import tpu as pltpu
from jax.experimental.pallas import tpu_sc as plsc

N = 10000
E = 160000
B = 2
P = 4
H = 8
NB = 8
M = N // 4
L = 128
PH = P * H
TILES = 16
CORES = 2
K = 125
CHUNKS = E // (TILES * K)
RPT = N // TILES
SLAB = 632
F32 = jnp.float32


def _sc_lgconv_body(zs_hbm, row_hbm, col_hbm, out_hbm,
                    rowv, colv, gbuf, zbuf, acc_sh, sem):
    c = lax.axis_index("c")
    s = lax.axis_index("s")

    pltpu.sync_copy(row_hbm.at[s], rowv)
    pltpu.sync_copy(col_hbm.at[s], colv)

    zvec = jnp.zeros((16,), F32)

    def _zb(i, carry):
        zbuf[i, pl.ds(0, 16)] = zvec
        zbuf[i, pl.ds(16, 16)] = zvec
        return carry

    lax.fori_loop(0, SLAB, _zb, 0)
    start = jnp.minimum(s * SLAB, N - SLAB)
    sl = pl.ds(start, SLAB)
    pltpu.sync_copy(zbuf, acc_sh.at[sl])
    plsc.subcore_barrier()

    zsb = zs_hbm.at[c]

    def _step(ch, carry):
        pltpu.async_copy(zsb.at[rowv.at[ch]], gbuf, sem).wait()
        pltpu.sync_copy(gbuf, acc_sh.at[colv.at[ch]], add=True)
        return carry

    lax.fori_loop(0, CHUNKS, _step, 0)
    plsc.subcore_barrier()

    pltpu.sync_copy(acc_sh.at[sl], out_hbm.at[c].at[sl])


@functools.cache
def _get_lgconv():
    return pl.kernel(
        _sc_lgconv_body,
        out_type=jax.ShapeDtypeStruct((CORES, N, PH), F32),
        mesh=plsc.VectorSubcoreMesh(core_axis_name="c", subcore_axis_name="s"),
        scratch_types=[
            pltpu.VMEM((CHUNKS, K), jnp.int32),
            pltpu.VMEM((CHUNKS, K), jnp.int32),
            pltpu.VMEM((K, PH), F32),
            pltpu.VMEM((SLAB, PH), F32),
            pltpu.VMEM_SHARED((N, PH), F32),
            pltpu.SemaphoreType.DMA,
        ],
        compiler_params=pltpu.CompilerParams(use_tc_tiling_on_sc=False),
    )


def _lgconv(zs2, row_r, col_r):
    return _get_lgconv()(zs2, row_r, col_r)


def _mm(x3, w):
    x2 = x3.reshape(-1, x3.shape[-1])
    y = lax.dot(x2, w, precision=lax.Precision.HIGHEST,
                preferred_element_type=F32)
    return y.reshape(x3.shape[:-1] + (w.shape[-1],))


def _fold_p():
    lane = lax.broadcasted_iota(jnp.int32, (L, P), 0)
    pcol = lax.broadcasted_iota(jnp.int32, (L, P), 1)
    return ((lane // H) % P == pcol).astype(F32)


def _ln(z, w, b):
    mp = _fold_p()
    cnt = float(N * H)
    cs = jnp.sum(z, axis=1)
    m = _mm(cs[:, None, :], mp)[:, 0, :] / cnt
    mb = _mm(m[:, None, :], mp.T)[:, 0, :]
    z0 = z - mb[:, None, :]
    s2 = jnp.sum(z0 * z0, axis=1)
    v = _mm(s2[:, None, :], mp)[:, 0, :] / cnt
    r = lax.rsqrt(v + 1e-5)
    rb = _mm(r[:, None, :], mp.T)[:, 0, :]
    return z0 * rb[:, None, :] * w + b


def _relu(x):
    return jnp.maximum(x, 0.0)


def _pre_body(xw, cond, pos_e, deg0, table, attn,
              p1W, p1b, p2W, p2b,
              w1r, b1r, kin2, b2r, kpos1, bp1r, kpos2, bp2r,
              wc1k, bc1r, wc2k, bc2r, gnw0, gnb0, time_s,
              zs_out, acc_out, dis_out, te_out):
    r0 = table[pl.ds(time_s[0], 1), :]
    r1 = table[pl.ds(time_s[1], 1), :]
    te = jnp.concatenate([r0, r1], axis=0)
    te = jax.nn.silu(_mm(te[:, None, :], p1W[...])[:, 0, :] + p1b[...])
    te = jax.nn.silu(_mm(te[:, None, :], p2W[...])[:, 0, :] + p2b[...])
    te_out[...] = te

    a = jax.nn.softmax(attn[...], axis=0)

    x = xw[...]
    h = _relu(x * w1r[...] + b1r[...])
    h = _mm(h, kin2[...]) + b2r[...]
    pp = _mm(_relu(_mm(pos_e[...][None], kpos1[...]) + bp1r[...]),
             kpos2[...]) + bp2r[...]
    h = h + pp
    acc_out[...] = h * a[0:1, :, None]

    c0 = _relu(_mm(cond[...], wc1k[...]) + bc1r[...])
    c0 = _mm(c0, wc2k[...]) + bc2r[...]
    zn = _ln(h + c0, gnw0[...], gnb0[...])

    dg = deg0[...]
    dis = jnp.where(dg > 0, lax.rsqrt(jnp.where(dg > 0, dg, 1.0)), 0.0)
    dis_out[...] = dis
    zs_out[...] = zn * dis


def _mid_body(g_in, cond, acc_in, dis, te, attn,
              wt1, bt1, wt2, bt2, tnw, tnb, tp, tcbr,
              wc1k, bc1r, wc2k, bc2r, gnw, gnb,
              zs_out, acc_out, *, jidx):
    a = jax.nn.softmax(attn[...], axis=0)
    tev = te[...]
    tb = _mm(_relu(_mm(tev[:, None, :], wt1[...])[:, 0, :] + bt1[...])[:, None, :],
             wt2[...])[:, 0, :] + bt2[...]
    tbrow = jnp.concatenate([tb] * (L // H), axis=1)

    disv = dis[...]
    z = g_in[...] * disv + tbrow[:, None, :]
    z = _ln(z, tnw[...], tnb[...])
    h = _mm(z, tp[...]) + tcbr[...]
    acc_out[...] = acc_in[...] + h * a[jidx + 1:jidx + 2, :, None]

    c = _relu(_mm(cond[...], wc1k[...]) + bc1r[...])
    c = _mm(c, wc2k[...]) + bc2r[...]
    zn = _ln(h + c, gnw[...], gnb[...])
    zs_out[...] = zn * disv


def _fin_body(g_in, acc_in, dis, te, attn,
              wt1, bt1, wt2, bt2, tnw, tnb, tp, tcbr,
              kout1, bo1r, kout2, bo2,
              o_out):
    a = jax.nn.softmax(attn[...], axis=0)
    tev = te[...]
    tb = _mm(_relu(_mm(tev[:, None, :], wt1[...])[:, 0, :] + bt1[...])[:, None, :],
             wt2[...])[:, 0, :] + bt2[...]
    tbrow = jnp.concatenate([tb] * (L // H), axis=1)

    z = g_in[...] * dis[...] + tbrow[:, None, :]
    z = _ln(z, tnw[...], tnb[...])
    h = _mm(z, tp[...]) + tcbr[...]
    acc = acc_in[...] + h * a[NB:NB + 1, :, None]

    u = _relu(_mm(acc, kout1[...]) + bo1r[...])
    o_out[...] = _mm(u, kout2[...]) + bo2[...]


def _tc_call(body, n_in, out_shapes, smem_last=False):
    specs = [pl.BlockSpec(memory_space=pltpu.VMEM) for _ in range(n_in)]
    if smem_last:
        specs[-1] = pl.BlockSpec(memory_space=pltpu.SMEM)
    return pl.pallas_call(
        body,
        in_specs=specs,
        out_specs=[pl.BlockSpec(memory_space=pltpu.VMEM)
                   for _ in range(len(out_shapes))],
        out_shape=out_shapes,
    )


def _tbl():
    steps = jnp.arange(500, dtype=F32)[:, None]
    dims = jnp.arange(16, dtype=F32)[None, :]
    e = steps * 10.0 ** (dims * 4.0 / 16.0)
    return jnp.concatenate([jnp.sin(e), jnp.cos(e)], axis=1)


def _kron(a, b):
    return jnp.einsum("ij,kl->ikjl", a, b).reshape(
        a.shape[0] * b.shape[0], a.shape[1] * b.shape[1])


def kernel(x, time, cond, pos, edge_index, p1W, p1b, p2W, p2b,
           inW1, inb1, inW2, inb2, posW1, posb1, posW2, posb2, attn,
           Wt1, bt1, Wt2, bt2, Wc1, bc1, Wc2, bc2, gnw, gnb, tnw, tnb,
           tcW, tcb, outW1, outb1, outW2, outb2):
    row = edge_index[0].astype(jnp.int32).reshape(TILES, CHUNKS, K)
    col = edge_index[1].astype(jnp.int32).reshape(TILES, CHUNKS, K)

    ones2 = jnp.ones((CORES, N, PH), F32)
    deg0 = _lgconv(ones2, row, col)[0].reshape(M, L)

    i8 = jnp.eye(H, dtype=F32)
    i16 = jnp.eye(16, dtype=F32)
    i4 = jnp.eye(4, dtype=F32)

    def _rowv(v):
        return jnp.tile(v, L // v.shape[0])[None]

    kin2 = _kron(i16, inW2)
    kpos1 = _kron(i16, posW1)
    kpos2 = _kron(i16, posW2)
    kout1 = _kron(i16, outW1)
    kout2 = _kron(i16, outW2)
    w1r = _rowv(inW1[0])
    b1r = _rowv(inb1)
    b2r = _rowv(inb2)
    bp1r = _rowv(posb1)
    bp2r = _rowv(posb2)
    bo1r = _rowv(outb1)
    bo2 = jnp.broadcast_to(outb2[None], (1, 16))

    xw = jnp.repeat(x.transpose(0, 2, 1), H, axis=-1).reshape(B, M, L)
    cond_t = cond.transpose(0, 2, 1, 3).reshape(B, M, L)
    pos_e = jnp.tile(pos, (1, P)).reshape(M, L)
    table = _tbl()

    def _blk(i):
        return dict(
            wc1k=_kron(i16, Wc1[i]), bc1r=_rowv(bc1[i]),
            wc2k=_kron(i16, Wc2[i]), bc2r=_rowv(bc2[i]),
            gnw=jnp.tile(gnw[i], (1, P)).reshape(M, L),
            gnb=jnp.tile(gnb[i], (1, P)).reshape(M, L),
            tnw=jnp.tile(tnw[i], (1, P)).reshape(M, L),
            tnb=jnp.tile(tnb[i], (1, P)).reshape(M, L),
            tp=_kron(i4, _kron(tcW[i].T, i8)),
            tcbr=jnp.tile(jnp.repeat(tcb[i], H), P)[None],
            wt1=Wt1[i], bt1=bt1[i][None], wt2=Wt2[i], bt2=bt2[i][None],
        )

    big = jax.ShapeDtypeStruct((B, M, L), F32)
    b0 = _blk(0)
    pre = _tc_call(_pre_body, 25, [
        big, big, jax.ShapeDtypeStruct((M, L), F32),
        jax.ShapeDtypeStruct((B, 64), F32)], smem_last=True)
    zs, acc, dis, te = pre(
        xw, cond_t, pos_e, deg0, table, attn,
        p1W, p1b[None], p2W, p2b[None],
        w1r, b1r, kin2, b2r, kpos1, bp1r, kpos2, bp2r,
        b0["wc1k"], b0["bc1r"], b0["wc2k"], b0["bc2r"],
        b0["gnw"], b0["gnb"], time.astype(jnp.int32))

    for j in range(NB - 1):
        g = _lgconv(zs.reshape(CORES, N, PH), row, col).reshape(B, M, L)
        bj = _blk(j)
        bn = _blk(j + 1)
        mid = _tc_call(functools.partial(_mid_body, jidx=j), 20, [big, big])
        zs, acc = mid(
            g, cond_t, acc, dis, te, attn,
            bj["wt1"], bj["bt1"], bj["wt2"], bj["bt2"],
            bj["tnw"], bj["tnb"], bj["tp"], bj["tcbr"],
            bn["wc1k"], bn["bc1r"], bn["wc2k"], bn["bc2r"],
            bn["gnw"], bn["gnb"])

    g = _lgconv(zs.reshape(CORES, N, PH), row, col).reshape(B, M, L)
    b7 = _blk(NB - 1)
    fin = _tc_call(_fin_body, 17, [jax.ShapeDtypeStruct((B, M, 16), F32)])
    o2, = fin(
        g, acc, dis, te, attn,
        b7["wt1"], b7["bt1"], b7["wt2"], b7["bt2"],
        b7["tnw"], b7["tnb"], b7["tp"], b7["tcbr"],
        kout1, bo1r, kout2, bo2)

    return o2.reshape(B, N, P).transpose(0, 2, 1)


# speedup vs baseline: 83.5749x; 1.0095x over previous
import functools

import jax
import jax.numpy as jnp
from jax import lax
from jax.experimental import pallas as pl
from jax.experimental.pallas import tpu as pltpu
from jax.experimental.pallas import tpu_sc as plsc

N = 10000
E = 160000
B = 2
P = 4
H = 8
NB = 8
M = N // 4
L = 128
PH = P * H
TILES = 16
CORES = 2
K = 125
CHUNKS = E // (TILES * K)
RPT = N // TILES
SLAB = 632
F32 = jnp.float32


NBUF = 4
LA = 2
GROUPS = CHUNKS // NBUF


def _sc_lgconv_body(zs_hbm, row_hbm, col_hbm, out_hbm,
                    rowv, colv, gb0, gb1, gb2, gb3, zbuf, acc_sh,
                    gsem, ssem):
    c = lax.axis_index("c")
    s = lax.axis_index("s")
    gbufs = (gb0, gb1, gb2, gb3)

    pltpu.sync_copy(row_hbm.at[s], rowv)
    pltpu.sync_copy(col_hbm.at[s], colv)

    zvec = jnp.zeros((16,), F32)

    def _zb(i, carry):
        zbuf[i, pl.ds(0, 16)] = zvec
        zbuf[i, pl.ds(16, 16)] = zvec
        return carry

    lax.fori_loop(0, SLAB, _zb, 0)
    start = jnp.minimum(s * SLAB, N - SLAB)
    sl = pl.ds(start, SLAB)
    pltpu.sync_copy(zbuf, acc_sh.at[sl])
    plsc.subcore_barrier()

    zsb = zs_hbm.at[c]

    def _gth(ch, b):
        return pltpu.make_async_copy(zsb.at[rowv.at[ch]], gbufs[b],
                                     gsem.at[b])

    def _sct(ch, b):
        return pltpu.make_async_copy(gbufs[b], acc_sh.at[colv.at[ch]],
                                     ssem.at[b])

    for b in range(LA):
        _gth(b, b).start()

    def _grp(gi, carry):
        for b in range(NBUF):
            ch = gi * NBUF + b
            _gth(ch, b).wait()
            pltpu.async_copy(gbufs[b], acc_sh.at[colv.at[ch]],
                             ssem.at[b], add=True)
            nb = (b + LA) % NBUF
            if b < NBUF - LA:
                @pl.when(gi > 0)
                def _():
                    _sct(ch - LA, nb).wait()
                _gth(ch + LA, nb).start()
            else:
                @pl.when(gi < GROUPS - 1)
                def _():
                    _sct(ch - LA, nb).wait()
                    _gth(ch + LA, nb).start()
        return carry

    lax.fori_loop(0, GROUPS, _grp, 0)
    for b in range(NBUF):
        _sct(CHUNKS - NBUF + b, b).wait()
    plsc.subcore_barrier()

    pltpu.sync_copy(acc_sh.at[sl], out_hbm.at[c].at[sl])


@functools.cache
def _get_lgconv():
    return pl.kernel(
        _sc_lgconv_body,
        out_type=jax.ShapeDtypeStruct((CORES, N, PH), F32),
        mesh=plsc.VectorSubcoreMesh(core_axis_name="c", subcore_axis_name="s"),
        scratch_types=[
            pltpu.VMEM((CHUNKS, K), jnp.int32),
            pltpu.VMEM((CHUNKS, K), jnp.int32),
            pltpu.VMEM((K, PH), F32),
            pltpu.VMEM((K, PH), F32),
            pltpu.VMEM((K, PH), F32),
            pltpu.VMEM((K, PH), F32),
            pltpu.VMEM((SLAB, PH), F32),
            pltpu.VMEM_SHARED((N, PH), F32),
            pltpu.SemaphoreType.DMA((NBUF,)),
            pltpu.SemaphoreType.DMA((NBUF,)),
        ],
        compiler_params=pltpu.CompilerParams(use_tc_tiling_on_sc=False),
    )


def _lgconv(zs2, row_r, col_r):
    return _get_lgconv()(zs2, row_r, col_r)


DCH = E // (CORES * TILES * K)


def _sc_deg_body(col_hbm, out_hbm, colv, ones, zbuf, acc_sh, ssem):
    c = lax.axis_index("c")
    s = lax.axis_index("s")

    pltpu.sync_copy(col_hbm.at[c].at[s], colv)

    one16 = jnp.full((16,), 1.0, F32)
    zvec = jnp.zeros((16,), F32)

    def _fill(i, carry):
        ones[i, pl.ds(0, 16)] = one16
        ones[i, pl.ds(16, 16)] = one16
        return carry

    lax.fori_loop(0, K, _fill, 0)

    def _zb(i, carry):
        zbuf[i, pl.ds(0, 16)] = zvec
        zbuf[i, pl.ds(16, 16)] = zvec
        return carry

    lax.fori_loop(0, SLAB, _zb, 0)
    start = jnp.minimum(s * SLAB, N - SLAB)
    sl = pl.ds(start, SLAB)
    pltpu.sync_copy(zbuf, acc_sh.at[sl])
    plsc.subcore_barrier()

    def _fire(ch, carry):
        pltpu.async_copy(ones, acc_sh.at[colv.at[ch]], ssem, add=True)
        return carry

    lax.fori_loop(0, DCH, _fire, 0)

    def _drain(ch, carry):
        pltpu.make_async_copy(ones, acc_sh.at[colv.at[ch]], ssem).wait()
        return carry

    lax.fori_loop(0, DCH, _drain, 0)
    plsc.subcore_barrier()
    pltpu.sync_copy(acc_sh.at[sl], out_hbm.at[c].at[sl])


@functools.cache
def _get_deg():
    return pl.kernel(
        _sc_deg_body,
        out_type=jax.ShapeDtypeStruct((CORES, N, PH), F32),
        mesh=plsc.VectorSubcoreMesh(core_axis_name="c", subcore_axis_name="s"),
        scratch_types=[
            pltpu.VMEM((DCH, K), jnp.int32),
            pltpu.VMEM((K, PH), F32),
            pltpu.VMEM((SLAB, PH), F32),
            pltpu.VMEM_SHARED((N, PH), F32),
            pltpu.SemaphoreType.DMA,
        ],
        compiler_params=pltpu.CompilerParams(use_tc_tiling_on_sc=False),
    )


def _mm(x3, w):
    x2 = x3.reshape(-1, x3.shape[-1])
    y = lax.dot(x2, w, precision=lax.Precision.HIGHEST,
                preferred_element_type=F32)
    return y.reshape(x3.shape[:-1] + (w.shape[-1],))


def _fold_p():
    lane = lax.broadcasted_iota(jnp.int32, (L, P), 0)
    pcol = lax.broadcasted_iota(jnp.int32, (L, P), 1)
    return ((lane // H) % P == pcol).astype(F32)


def _ln(z, w, b):
    mp = _fold_p()
    cnt = float(N * H)
    cs = jnp.sum(z, axis=1)
    m = _mm(cs[:, None, :], mp)[:, 0, :] / cnt
    mb = _mm(m[:, None, :], mp.T)[:, 0, :]
    z0 = z - mb[:, None, :]
    s2 = jnp.sum(z0 * z0, axis=1)
    v = _mm(s2[:, None, :], mp)[:, 0, :] / cnt
    r = lax.rsqrt(v + 1e-5)
    rb = _mm(r[:, None, :], mp.T)[:, 0, :]
    return z0 * rb[:, None, :] * w + b


def _relu(x):
    return jnp.maximum(x, 0.0)


def _pre_body(xw, cond, pos_e, deg2, table, attn,
              p1W, p1b, p2W, p2b,
              w1r, b1r, kin2, b2r, kpos1, bp1r, kpos2, bp2r,
              wc1k, bc1r, wc2k, bc2r, gnw0, gnb0, time_s,
              zs_out, acc_out, dis_out, te_out):
    r0 = table[pl.ds(time_s[0], 1), :]
    r1 = table[pl.ds(time_s[1], 1), :]
    te = jnp.concatenate([r0, r1], axis=0)
    te = jax.nn.silu(_mm(te[:, None, :], p1W[...])[:, 0, :] + p1b[...])
    te = jax.nn.silu(_mm(te[:, None, :], p2W[...])[:, 0, :] + p2b[...])
    te_out[...] = te

    a = jax.nn.softmax(attn[...], axis=0)

    x = xw[...]
    h = _relu(x * w1r[...] + b1r[...])
    h = _mm(h, kin2[...]) + b2r[...]
    pp = _mm(_relu(_mm(pos_e[...][None], kpos1[...]) + bp1r[...]),
             kpos2[...]) + bp2r[...]
    h = h + pp
    acc_out[...] = h * a[0:1, :, None]

    c0 = _relu(_mm(cond[...], wc1k[...]) + bc1r[...])
    c0 = _mm(c0, wc2k[...]) + bc2r[...]
    zn = _ln(h + c0, gnw0[...], gnb0[...])

    dgv = deg2[...]
    dg = dgv[0] + dgv[1]
    dis = jnp.where(dg > 0, lax.rsqrt(jnp.where(dg > 0, dg, 1.0)), 0.0)
    dis_out[...] = dis
    zs_out[...] = zn * dis


def _mid_body(g_in, cond, acc_in, dis, te, attn,
              wt1, bt1, wt2, bt2, tnw, tnb, tp, tcbr,
              wc1k, bc1r, wc2k, bc2r, gnw, gnb,
              zs_out, acc_out, *, jidx):
    a = jax.nn.softmax(attn[...], axis=0)
    tev = te[...]
    tb = _mm(_relu(_mm(tev[:, None, :], wt1[...])[:, 0, :] + bt1[...])[:, None, :],
             wt2[...])[:, 0, :] + bt2[...]
    tbrow = jnp.concatenate([tb] * (L // H), axis=1)

    disv = dis[...]
    z = g_in[...] * disv + tbrow[:, None, :]
    z = _ln(z, tnw[...], tnb[...])
    h = _mm(z, tp[...]) + tcbr[...]
    acc_out[...] = acc_in[...] + h * a[jidx + 1:jidx + 2, :, None]

    c = _relu(_mm(cond[...], wc1k[...]) + bc1r[...])
    c = _mm(c, wc2k[...]) + bc2r[...]
    zn = _ln(h + c, gnw[...], gnb[...])
    zs_out[...] = zn * disv


def _fin_body(g_in, acc_in, dis, te, attn,
              wt1, bt1, wt2, bt2, tnw, tnb, tp, tcbr,
              kout1, bo1r, kout2, bo2,
              o_out):
    a = jax.nn.softmax(attn[...], axis=0)
    tev = te[...]
    tb = _mm(_relu(_mm(tev[:, None, :], wt1[...])[:, 0, :] + bt1[...])[:, None, :],
             wt2[...])[:, 0, :] + bt2[...]
    tbrow = jnp.concatenate([tb] * (L // H), axis=1)

    z = g_in[...] * dis[...] + tbrow[:, None, :]
    z = _ln(z, tnw[...], tnb[...])
    h = _mm(z, tp[...]) + tcbr[...]
    acc = acc_in[...] + h * a[NB:NB + 1, :, None]

    u = _relu(_mm(acc, kout1[...]) + bo1r[...])
    o_out[...] = _mm(u, kout2[...]) + bo2[...]


def _tc_call(body, n_in, out_shapes, smem_last=False):
    specs = [pl.BlockSpec(memory_space=pltpu.VMEM) for _ in range(n_in)]
    if smem_last:
        specs[-1] = pl.BlockSpec(memory_space=pltpu.SMEM)
    return pl.pallas_call(
        body,
        in_specs=specs,
        out_specs=[pl.BlockSpec(memory_space=pltpu.VMEM)
                   for _ in range(len(out_shapes))],
        out_shape=out_shapes,
    )


def _tbl():
    steps = jnp.arange(500, dtype=F32)[:, None]
    dims = jnp.arange(16, dtype=F32)[None, :]
    e = steps * 10.0 ** (dims * 4.0 / 16.0)
    return jnp.concatenate([jnp.sin(e), jnp.cos(e)], axis=1)


def _kron(a, b):
    return jnp.einsum("ij,kl->ikjl", a, b).reshape(
        a.shape[0] * b.shape[0], a.shape[1] * b.shape[1])


def kernel(x, time, cond, pos, edge_index, p1W, p1b, p2W, p2b,
           inW1, inb1, inW2, inb2, posW1, posb1, posW2, posb2, attn,
           Wt1, bt1, Wt2, bt2, Wc1, bc1, Wc2, bc2, gnw, gnb, tnw, tnb,
           tcW, tcb, outW1, outb1, outW2, outb2):
    row = edge_index[0].astype(jnp.int32).reshape(TILES, CHUNKS, K)
    col = edge_index[1].astype(jnp.int32).reshape(TILES, CHUNKS, K)

    col_d = edge_index[1].astype(jnp.int32).reshape(CORES, TILES, DCH, K)
    deg2 = _get_deg()(col_d).reshape(CORES, M, L)

    i8 = jnp.eye(H, dtype=F32)
    i16 = jnp.eye(16, dtype=F32)
    i4 = jnp.eye(4, dtype=F32)

    def _rowv(v):
        return jnp.tile(v, L // v.shape[0])[None]

    kin2 = _kron(i16, inW2)
    kpos1 = _kron(i16, posW1)
    kpos2 = _kron(i16, posW2)
    kout1 = _kron(i16, outW1)
    kout2 = _kron(i16, outW2)
    w1r = _rowv(inW1[0])
    b1r = _rowv(inb1)
    b2r = _rowv(inb2)
    bp1r = _rowv(posb1)
    bp2r = _rowv(posb2)
    bo1r = _rowv(outb1)
    bo2 = jnp.broadcast_to(outb2[None], (1, 16))

    xw = jnp.repeat(x.transpose(0, 2, 1), H, axis=-1).reshape(B, M, L)
    cond_t = cond.transpose(0, 2, 1, 3).reshape(B, M, L)
    pos_e = jnp.tile(pos, (1, P)).reshape(M, L)
    table = _tbl()

    def _blk(i):
        return dict(
            wc1k=_kron(i16, Wc1[i]), bc1r=_rowv(bc1[i]),
            wc2k=_kron(i16, Wc2[i]), bc2r=_rowv(bc2[i]),
            gnw=jnp.tile(gnw[i], (1, P)).reshape(M, L),
            gnb=jnp.tile(gnb[i], (1, P)).reshape(M, L),
            tnw=jnp.tile(tnw[i], (1, P)).reshape(M, L),
            tnb=jnp.tile(tnb[i], (1, P)).reshape(M, L),
            tp=_kron(i4, _kron(tcW[i].T, i8)),
            tcbr=jnp.tile(jnp.repeat(tcb[i], H), P)[None],
            wt1=Wt1[i], bt1=bt1[i][None], wt2=Wt2[i], bt2=bt2[i][None],
        )

    big = jax.ShapeDtypeStruct((B, M, L), F32)
    b0 = _blk(0)
    pre = _tc_call(_pre_body, 25, [
        big, big, jax.ShapeDtypeStruct((M, L), F32),
        jax.ShapeDtypeStruct((B, 64), F32)], smem_last=True)
    zs, acc, dis, te = pre(
        xw, cond_t, pos_e, deg2, table, attn,
        p1W, p1b[None], p2W, p2b[None],
        w1r, b1r, kin2, b2r, kpos1, bp1r, kpos2, bp2r,
        b0["wc1k"], b0["bc1r"], b0["wc2k"], b0["bc2r"],
        b0["gnw"], b0["gnb"], time.astype(jnp.int32))

    for j in range(NB - 1):
        g = _lgconv(zs.reshape(CORES, N, PH), row, col).reshape(B, M, L)
        bj = _blk(j)
        bn = _blk(j + 1)
        mid = _tc_call(functools.partial(_mid_body, jidx=j), 20, [big, big])
        zs, acc = mid(
            g, cond_t, acc, dis, te, attn,
            bj["wt1"], bj["bt1"], bj["wt2"], bj["bt2"],
            bj["tnw"], bj["tnb"], bj["tp"], bj["tcbr"],
            bn["wc1k"], bn["bc1r"], bn["wc2k"], bn["bc2r"],
            bn["gnw"], bn["gnb"])

    g = _lgconv(zs.reshape(CORES, N, PH), row, col).reshape(B, M, L)
    b7 = _blk(NB - 1)
    fin = _tc_call(_fin_body, 17, [jax.ShapeDtypeStruct((B, M, 16), F32)])
    o2, = fin(
        g, acc, dis, te, attn,
        b7["wt1"], b7["bt1"], b7["wt2"], b7["bt2"],
        b7["tnw"], b7["tnb"], b7["tp"], b7["tcbr"],
        kout1, bo1r, kout2, bo2)

    return o2.reshape(B, N, P).transpose(0, 2, 1)


# speedup vs baseline: 167.6000x; 2.0054x over previous
import functools

import jax
import jax.numpy as jnp
from jax import lax
from jax.experimental import pallas as pl
from jax.experimental.pallas import tpu as pltpu
from jax.experimental.pallas import tpu_sc as plsc

N = 10000
E = 160000
B = 2
P = 4
H = 8
NB = 8
M = N // 4
L = 128
PH = P * H
TILES = 16
CORES = 2
K = 125
CHUNKS = E // (TILES * K)
RPT = N // TILES
SLAB = 632
F32 = jnp.float32


NBUF = 4
LA = 2
GROUPS = CHUNKS // NBUF


def _sc_lgconv_body(zs_hbm, row_hbm, col_hbm, out_hbm,
                    rowv, colv, gb0, gb1, gb2, gb3, zbuf, acc_sh,
                    gsem, ssem):
    c = lax.axis_index("c")
    s = lax.axis_index("s")
    gbufs = (gb0, gb1, gb2, gb3)

    pltpu.sync_copy(row_hbm.at[s], rowv)
    pltpu.sync_copy(col_hbm.at[s], colv)

    zvec = jnp.zeros((16,), F32)

    def _zb(i, carry):
        zbuf[i, pl.ds(0, 16)] = zvec
        zbuf[i, pl.ds(16, 16)] = zvec
        return carry

    lax.fori_loop(0, SLAB, _zb, 0)
    start = jnp.minimum(s * SLAB, N - SLAB)
    sl = pl.ds(start, SLAB)
    pltpu.sync_copy(zbuf, acc_sh.at[sl])
    plsc.subcore_barrier()

    zsb = zs_hbm.at[c]

    def _gth(ch, b):
        return pltpu.make_async_copy(zsb.at[rowv.at[ch]], gbufs[b],
                                     gsem.at[b])

    def _sct(ch, b):
        return pltpu.make_async_copy(gbufs[b], acc_sh.at[colv.at[ch]],
                                     ssem.at[b])

    for b in range(LA):
        _gth(b, b).start()

    def _grp(gi, carry):
        for b in range(NBUF):
            ch = gi * NBUF + b
            _gth(ch, b).wait()
            pltpu.async_copy(gbufs[b], acc_sh.at[colv.at[ch]],
                             ssem.at[b], add=True)
            nb = (b + LA) % NBUF
            if b < NBUF - LA:
                @pl.when(gi > 0)
                def _():
                    _sct(ch - LA, nb).wait()
                _gth(ch + LA, nb).start()
            else:
                @pl.when(gi < GROUPS - 1)
                def _():
                    _sct(ch - LA, nb).wait()
                    _gth(ch + LA, nb).start()
        return carry

    lax.fori_loop(0, GROUPS, _grp, 0)
    for b in range(NBUF):
        _sct(CHUNKS - NBUF + b, b).wait()
    plsc.subcore_barrier()

    pltpu.sync_copy(acc_sh.at[sl], out_hbm.at[c].at[sl])


@functools.cache
def _get_lgconv():
    return pl.kernel(
        _sc_lgconv_body,
        out_type=jax.ShapeDtypeStruct((CORES, N, PH), F32),
        mesh=plsc.VectorSubcoreMesh(core_axis_name="c", subcore_axis_name="s"),
        scratch_types=[
            pltpu.VMEM((CHUNKS, K), jnp.int32),
            pltpu.VMEM((CHUNKS, K), jnp.int32),
            pltpu.VMEM((K, PH), F32),
            pltpu.VMEM((K, PH), F32),
            pltpu.VMEM((K, PH), F32),
            pltpu.VMEM((K, PH), F32),
            pltpu.VMEM((SLAB, PH), F32),
            pltpu.VMEM_SHARED((N, PH), F32),
            pltpu.SemaphoreType.DMA((NBUF,)),
            pltpu.SemaphoreType.DMA((NBUF,)),
        ],
        compiler_params=pltpu.CompilerParams(use_tc_tiling_on_sc=False),
    )


def _lgconv(zs2, row_r, col_r):
    return _get_lgconv()(zs2, row_r, col_r)


DCH = E // (CORES * TILES * K)


def _sc_deg_body(col_hbm, out_hbm, colv, ones, zbuf, acc_sh, ssem):
    c = lax.axis_index("c")
    s = lax.axis_index("s")

    pltpu.sync_copy(col_hbm.at[c].at[s], colv)

    one16 = jnp.full((16,), 1.0, F32)
    zvec = jnp.zeros((16,), F32)

    def _fill(i, carry):
        ones[i, pl.ds(0, 16)] = one16
        ones[i, pl.ds(16, 16)] = one16
        return carry

    lax.fori_loop(0, K, _fill, 0)

    def _zb(i, carry):
        zbuf[i, pl.ds(0, 16)] = zvec
        zbuf[i, pl.ds(16, 16)] = zvec
        return carry

    lax.fori_loop(0, SLAB, _zb, 0)
    start = jnp.minimum(s * SLAB, N - SLAB)
    sl = pl.ds(start, SLAB)
    pltpu.sync_copy(zbuf, acc_sh.at[sl])
    plsc.subcore_barrier()

    def _fire(ch, carry):
        pltpu.async_copy(ones, acc_sh.at[colv.at[ch]], ssem, add=True)
        return carry

    lax.fori_loop(0, DCH, _fire, 0)

    def _drain(ch, carry):
        pltpu.make_async_copy(ones, acc_sh.at[colv.at[ch]], ssem).wait()
        return carry

    lax.fori_loop(0, DCH, _drain, 0)
    plsc.subcore_barrier()
    pltpu.sync_copy(acc_sh.at[sl], out_hbm.at[c].at[sl])


@functools.cache
def _get_deg():
    return pl.kernel(
        _sc_deg_body,
        out_type=jax.ShapeDtypeStruct((CORES, N, PH), F32),
        mesh=plsc.VectorSubcoreMesh(core_axis_name="c", subcore_axis_name="s"),
        scratch_types=[
            pltpu.VMEM((DCH, K), jnp.int32),
            pltpu.VMEM((K, PH), F32),
            pltpu.VMEM((SLAB, PH), F32),
            pltpu.VMEM_SHARED((N, PH), F32),
            pltpu.SemaphoreType.DMA,
        ],
        compiler_params=pltpu.CompilerParams(use_tc_tiling_on_sc=False),
    )


def _mm(x3, w):
    x2 = x3.reshape(-1, x3.shape[-1])
    y = lax.dot(x2, w, precision=lax.Precision.HIGHEST,
                preferred_element_type=F32)
    return y.reshape(x3.shape[:-1] + (w.shape[-1],))


def _fold_p():
    lane = lax.broadcasted_iota(jnp.int32, (L, P), 0)
    pcol = lax.broadcasted_iota(jnp.int32, (L, P), 1)
    return ((lane // H) % P == pcol).astype(F32)


def _sp32():
    u = lax.broadcasted_iota(jnp.int32, (PH, L), 0)
    l = lax.broadcasted_iota(jnp.int32, (PH, L), 1)
    return ((l % H == u % H) & (l // PH == u // H)).astype(F32)


def _exb(b):
    u = lax.broadcasted_iota(jnp.int32, (PH, L), 0)
    l = lax.broadcasted_iota(jnp.int32, (PH, L), 1)
    keep = ((u // H == l // PH) & ((u % H) % P == (l // H) % P)
            & ((u % H) // P == b))
    return keep.astype(F32)


def _ln(z, w, b):
    mp = _fold_p()
    cnt = float(N * H)
    cs = jnp.sum(z, axis=1)
    m = _mm(cs[:, None, :], mp)[:, 0, :] / cnt
    mb = _mm(m[:, None, :], mp.T)[:, 0, :]
    z0 = z - mb[:, None, :]
    s2 = jnp.sum(z0 * z0, axis=1)
    v = _mm(s2[:, None, :], mp)[:, 0, :] / cnt
    r = lax.rsqrt(v + 1e-5)
    rb = _mm(r[:, None, :], mp.T)[:, 0, :]
    return z0 * rb[:, None, :] * w + b


def _relu(x):
    return jnp.maximum(x, 0.0)


def _pre_body(x8, cond, pos32, deg2, table, attn,
              p1W, p1b, p2W, p2b,
              w1r, b1r, kin2, b2r, kp1, bp1r, kp2, bp2r,
              wc1k, bc1r, wc2k, bc2r, gnw0, gnb0, time_s,
              zs_out, acc_out, dis_out, te_out):
    r0 = table[pl.ds(time_s[0], 1), :]
    r1 = table[pl.ds(time_s[1], 1), :]
    te = jnp.concatenate([r0, r1], axis=0)
    te = jax.nn.silu(_mm(te[:, None, :], p1W[...])[:, 0, :] + p1b[...])
    te = jax.nn.silu(_mm(te[:, None, :], p2W[...])[:, 0, :] + p2b[...])
    te_out[...] = te

    a = jax.nn.softmax(attn[...], axis=0)
    sp = _sp32()

    xt = x8[...]
    xw = jnp.stack([_mm(xt, _exb(0)), _mm(xt, _exb(1))])
    h = _relu(xw * w1r[...] + b1r[...])
    h = _mm(h, kin2[...]) + b2r[...]
    pp = _relu(_mm(pos32[...], kp1[...]) + bp1r[...])
    pp = _mm(pp, kp2[...]) + bp2r[...]
    h = h + _mm(pp, sp)
    acc_out[...] = h * a[0:1, :, None]

    c0 = _relu(_mm(cond[...], wc1k[...]) + bc1r[...])
    c0 = _mm(c0, wc2k[...]) + bc2r[...]
    zn = _ln(h + c0, _mm(gnw0[...], sp), _mm(gnb0[...], sp))

    dgv = deg2[...]
    dg = dgv[0] + dgv[1]
    dis = jnp.where(dg > 0, lax.rsqrt(jnp.where(dg > 0, dg, 1.0)), 0.0)
    dis_out[...] = dis
    zs_out[...] = zn * dis


def _mid_body(g_in, cond, acc_in, dis, te, attn,
              wt1, bt1, wt2, bt2, tnw, tnb, tp, tcbr,
              wc1k, bc1r, wc2k, bc2r, gnw, gnb,
              zs_out, acc_out, *, jidx):
    a = jax.nn.softmax(attn[...], axis=0)
    tev = te[...]
    tb = _mm(_relu(_mm(tev[:, None, :], wt1[...])[:, 0, :] + bt1[...])[:, None, :],
             wt2[...])[:, 0, :] + bt2[...]
    tbrow = jnp.concatenate([tb] * (L // H), axis=1)

    sp = _sp32()
    disv = dis[...]
    z = g_in[...] * disv + tbrow[:, None, :]
    z = _ln(z, _mm(tnw[...], sp), _mm(tnb[...], sp))
    h = _mm(z, tp[...]) + tcbr[...]
    acc_out[...] = acc_in[...] + h * a[jidx + 1:jidx + 2, :, None]

    c = _relu(_mm(cond[...], wc1k[...]) + bc1r[...])
    c = _mm(c, wc2k[...]) + bc2r[...]
    zn = _ln(h + c, _mm(gnw[...], sp), _mm(gnb[...], sp))
    zs_out[...] = zn * disv


def _fin_body(g_in, acc_in, dis, te, attn,
              wt1, bt1, wt2, bt2, tnw, tnb, tp, tcbr,
              kout1, bo1r, kout2, bo2,
              o_out):
    a = jax.nn.softmax(attn[...], axis=0)
    tev = te[...]
    tb = _mm(_relu(_mm(tev[:, None, :], wt1[...])[:, 0, :] + bt1[...])[:, None, :],
             wt2[...])[:, 0, :] + bt2[...]
    tbrow = jnp.concatenate([tb] * (L // H), axis=1)

    sp = _sp32()
    z = g_in[...] * dis[...] + tbrow[:, None, :]
    z = _ln(z, _mm(tnw[...], sp), _mm(tnb[...], sp))
    h = _mm(z, tp[...]) + tcbr[...]
    acc = acc_in[...] + h * a[NB:NB + 1, :, None]

    u = _relu(_mm(acc, kout1[...]) + bo1r[...])
    o_out[...] = _mm(u, kout2[...]) + bo2[...]


def _tc_call(body, n_in, out_shapes, smem_last=False):
    specs = [pl.BlockSpec(memory_space=pltpu.VMEM) for _ in range(n_in)]
    if smem_last:
        specs[-1] = pl.BlockSpec(memory_space=pltpu.SMEM)
    return pl.pallas_call(
        body,
        in_specs=specs,
        out_specs=[pl.BlockSpec(memory_space=pltpu.VMEM)
                   for _ in range(len(out_shapes))],
        out_shape=out_shapes,
    )


def _tbl():
    steps = jnp.arange(500, dtype=F32)[:, None]
    dims = jnp.arange(16, dtype=F32)[None, :]
    e = steps * 10.0 ** (dims * 4.0 / 16.0)
    return jnp.concatenate([jnp.sin(e), jnp.cos(e)], axis=1)


def _kron(a, b):
    return jnp.einsum("ij,kl->ikjl", a, b).reshape(
        a.shape[0] * b.shape[0], a.shape[1] * b.shape[1])


def kernel(x, time, cond, pos, edge_index, p1W, p1b, p2W, p2b,
           inW1, inb1, inW2, inb2, posW1, posb1, posW2, posb2, attn,
           Wt1, bt1, Wt2, bt2, Wc1, bc1, Wc2, bc2, gnw, gnb, tnw, tnb,
           tcW, tcb, outW1, outb1, outW2, outb2):
    row = edge_index[0].astype(jnp.int32).reshape(TILES, CHUNKS, K)
    col = edge_index[1].astype(jnp.int32).reshape(TILES, CHUNKS, K)

    col_d = edge_index[1].astype(jnp.int32).reshape(CORES, TILES, DCH, K)
    deg2 = _get_deg()(col_d).reshape(CORES, M, L)

    i8 = jnp.eye(H, dtype=F32)
    i16 = jnp.eye(16, dtype=F32)
    i4 = jnp.eye(4, dtype=F32)

    def _rowv(v):
        return jnp.tile(v, L // v.shape[0])[None]

    kin2 = _kron(i16, inW2)
    kp1 = _kron(i4, posW1)
    kp2 = _kron(i4, posW2)
    kout1 = _kron(i16, outW1)
    kout2 = _kron(i16, outW2)
    w1r = _rowv(inW1[0])
    b1r = _rowv(inb1)
    b2r = _rowv(inb2)
    bp1r = jnp.tile(posb1, P)[None]
    bp2r = jnp.tile(posb2, P)[None]
    bo1r = _rowv(outb1)
    bo2 = jnp.broadcast_to(outb2[None], (1, 16))

    x8 = x.reshape(B * P, M, 4).transpose(1, 2, 0).reshape(M, PH)
    cond_t = cond.transpose(0, 2, 1, 3).reshape(B, M, L)
    pos32 = pos.reshape(M, PH)
    table = _tbl()

    def _blk(i):
        return dict(
            wc1k=_kron(i16, Wc1[i]), bc1r=_rowv(bc1[i]),
            wc2k=_kron(i16, Wc2[i]), bc2r=_rowv(bc2[i]),
            gnw=gnw[i].reshape(M, PH),
            gnb=gnb[i].reshape(M, PH),
            tnw=tnw[i].reshape(M, PH),
            tnb=tnb[i].reshape(M, PH),
            tp=_kron(i4, _kron(tcW[i].T, i8)),
            tcbr=jnp.tile(jnp.repeat(tcb[i], H), P)[None],
            wt1=Wt1[i], bt1=bt1[i][None], wt2=Wt2[i], bt2=bt2[i][None],
        )

    big = jax.ShapeDtypeStruct((B, M, L), F32)
    b0 = _blk(0)
    pre = _tc_call(_pre_body, 25, [
        big, big, jax.ShapeDtypeStruct((M, L), F32),
        jax.ShapeDtypeStruct((B, 64), F32)], smem_last=True)
    zs, acc, dis, te = pre(
        x8, cond_t, pos32, deg2, table, attn,
        p1W, p1b[None], p2W, p2b[None],
        w1r, b1r, kin2, b2r, kp1, bp1r, kp2, bp2r,
        b0["wc1k"], b0["bc1r"], b0["wc2k"], b0["bc2r"],
        b0["gnw"], b0["gnb"], time.astype(jnp.int32))

    for j in range(NB - 1):
        g = _lgconv(zs.reshape(CORES, N, PH), row, col).reshape(B, M, L)
        bj = _blk(j)
        bn = _blk(j + 1)
        mid = _tc_call(functools.partial(_mid_body, jidx=j), 20, [big, big])
        zs, acc = mid(
            g, cond_t, acc, dis, te, attn,
            bj["wt1"], bj["bt1"], bj["wt2"], bj["bt2"],
            bj["tnw"], bj["tnb"], bj["tp"], bj["tcbr"],
            bn["wc1k"], bn["bc1r"], bn["wc2k"], bn["bc2r"],
            bn["gnw"], bn["gnb"])

    g = _lgconv(zs.reshape(CORES, N, PH), row, col).reshape(B, M, L)
    b7 = _blk(NB - 1)
    fin = _tc_call(_fin_body, 17, [jax.ShapeDtypeStruct((B, M, 16), F32)])
    o2, = fin(
        g, acc, dis, te, attn,
        b7["wt1"], b7["bt1"], b7["wt2"], b7["bt2"],
        b7["tnw"], b7["tnb"], b7["tp"], b7["tcbr"],
        kout1, bo1r, kout2, bo2)

    return o2.reshape(B, N, P).transpose(0, 2, 1)


# speedup vs baseline: 183.3795x; 1.0942x over previous
import functools

import jax
import jax.numpy as jnp
from jax import lax
from jax.experimental import pallas as pl
from jax.experimental.pallas import tpu as pltpu
from jax.experimental.pallas import tpu_sc as plsc

N = 10000
E = 160000
B = 2
P = 4
H = 8
NB = 8
M = N // 4
L = 128
PH = P * H
TILES = 16
CORES = 2
K = 125
CHUNKS = E // (TILES * K)
RPT = N // TILES
SLAB = 632
F32 = jnp.float32


NBUF = 4
LA = 2
GROUPS = CHUNKS // NBUF


def _sc_lgconv_body(zs_hbm, row_hbm, col_hbm, out_hbm,
                    rowv, colv, gb0, gb1, gb2, gb3, zbuf, acc_sh, in_sh,
                    gsem, ssem):
    c = lax.axis_index("c")
    s = lax.axis_index("s")
    gbufs = (gb0, gb1, gb2, gb3)

    pltpu.sync_copy(row_hbm.at[s], rowv)
    pltpu.sync_copy(col_hbm.at[s], colv)

    zvec = jnp.zeros((16,), F32)

    def _zb(i, carry):
        zbuf[i, pl.ds(0, 16)] = zvec
        zbuf[i, pl.ds(16, 16)] = zvec
        return carry

    lax.fori_loop(0, SLAB, _zb, 0)
    start = jnp.minimum(s * SLAB, N - SLAB)
    sl = pl.ds(start, SLAB)
    pltpu.sync_copy(zbuf, acc_sh.at[sl])
    pltpu.sync_copy(zs_hbm.at[c].at[sl], in_sh.at[sl])
    plsc.subcore_barrier()

    zsb = in_sh

    def _gth(ch, b):
        return pltpu.make_async_copy(zsb.at[rowv.at[ch]], gbufs[b],
                                     gsem.at[b])

    def _sct(ch, b):
        return pltpu.make_async_copy(gbufs[b], acc_sh.at[colv.at[ch]],
                                     ssem.at[b])

    for b in range(LA):
        _gth(b, b).start()

    def _grp(gi, carry):
        for b in range(NBUF):
            ch = gi * NBUF + b
            _gth(ch, b).wait()
            pltpu.async_copy(gbufs[b], acc_sh.at[colv.at[ch]],
                             ssem.at[b], add=True)
            nb = (b + LA) % NBUF
            if b < NBUF - LA:
                @pl.when(gi > 0)
                def _():
                    _sct(ch - LA, nb).wait()
                _gth(ch + LA, nb).start()
            else:
                @pl.when(gi < GROUPS - 1)
                def _():
                    _sct(ch - LA, nb).wait()
                    _gth(ch + LA, nb).start()
        return carry

    lax.fori_loop(0, GROUPS, _grp, 0)
    for b in range(NBUF):
        _sct(CHUNKS - NBUF + b, b).wait()
    plsc.subcore_barrier()

    pltpu.sync_copy(acc_sh.at[sl], out_hbm.at[c].at[sl])


@functools.cache
def _get_lgconv():
    return pl.kernel(
        _sc_lgconv_body,
        out_type=jax.ShapeDtypeStruct((CORES, N, PH), F32),
        mesh=plsc.VectorSubcoreMesh(core_axis_name="c", subcore_axis_name="s"),
        scratch_types=[
            pltpu.VMEM((CHUNKS, K), jnp.int32),
            pltpu.VMEM((CHUNKS, K), jnp.int32),
            pltpu.VMEM((K, PH), F32),
            pltpu.VMEM((K, PH), F32),
            pltpu.VMEM((K, PH), F32),
            pltpu.VMEM((K, PH), F32),
            pltpu.VMEM((SLAB, PH), F32),
            pltpu.VMEM_SHARED((N, PH), F32),
            pltpu.VMEM_SHARED((N, PH), F32),
            pltpu.SemaphoreType.DMA((NBUF,)),
            pltpu.SemaphoreType.DMA((NBUF,)),
        ],
        compiler_params=pltpu.CompilerParams(use_tc_tiling_on_sc=False),
    )


def _lgconv(zs2, row_r, col_r):
    return _get_lgconv()(zs2, row_r, col_r)


DCH = E // (CORES * TILES * K)


def _sc_deg_body(col_hbm, out_hbm, colv, ones, zbuf, acc_sh, ssem):
    c = lax.axis_index("c")
    s = lax.axis_index("s")

    pltpu.sync_copy(col_hbm.at[c].at[s], colv)

    one16 = jnp.full((16,), 1.0, F32)
    zvec = jnp.zeros((16,), F32)

    def _fill(i, carry):
        ones[i, pl.ds(0, 16)] = one16
        ones[i, pl.ds(16, 16)] = one16
        return carry

    lax.fori_loop(0, K, _fill, 0)

    def _zb(i, carry):
        zbuf[i, pl.ds(0, 16)] = zvec
        zbuf[i, pl.ds(16, 16)] = zvec
        return carry

    lax.fori_loop(0, SLAB, _zb, 0)
    start = jnp.minimum(s * SLAB, N - SLAB)
    sl = pl.ds(start, SLAB)
    pltpu.sync_copy(zbuf, acc_sh.at[sl])
    plsc.subcore_barrier()

    def _fire(ch, carry):
        pltpu.async_copy(ones, acc_sh.at[colv.at[ch]], ssem, add=True)
        return carry

    lax.fori_loop(0, DCH, _fire, 0)

    def _drain(ch, carry):
        pltpu.make_async_copy(ones, acc_sh.at[colv.at[ch]], ssem).wait()
        return carry

    lax.fori_loop(0, DCH, _drain, 0)
    plsc.subcore_barrier()
    pltpu.sync_copy(acc_sh.at[sl], out_hbm.at[c].at[sl])


@functools.cache
def _get_deg():
    return pl.kernel(
        _sc_deg_body,
        out_type=jax.ShapeDtypeStruct((CORES, N, PH), F32),
        mesh=plsc.VectorSubcoreMesh(core_axis_name="c", subcore_axis_name="s"),
        scratch_types=[
            pltpu.VMEM((DCH, K), jnp.int32),
            pltpu.VMEM((K, PH), F32),
            pltpu.VMEM((SLAB, PH), F32),
            pltpu.VMEM_SHARED((N, PH), F32),
            pltpu.SemaphoreType.DMA,
        ],
        compiler_params=pltpu.CompilerParams(use_tc_tiling_on_sc=False),
    )


def _mm(x3, w):
    x2 = x3.reshape(-1, x3.shape[-1])
    y = lax.dot(x2, w, precision=lax.Precision.HIGHEST,
                preferred_element_type=F32)
    return y.reshape(x3.shape[:-1] + (w.shape[-1],))


def _fold_p():
    lane = lax.broadcasted_iota(jnp.int32, (L, P), 0)
    pcol = lax.broadcasted_iota(jnp.int32, (L, P), 1)
    return ((lane // H) % P == pcol).astype(F32)


def _sp32():
    u = lax.broadcasted_iota(jnp.int32, (PH, L), 0)
    l = lax.broadcasted_iota(jnp.int32, (PH, L), 1)
    return ((l % H == u % H) & (l // PH == u // H)).astype(F32)


def _rq(q):
    u = lax.broadcasted_iota(jnp.int32, (P * L, L), 0)
    lo = lax.broadcasted_iota(jnp.int32, (P * L, L), 1)
    p = u // L
    li = u % L
    keep = ((lo % H == li % H) & (li // H == P * q + lo // PH)
            & ((lo // H) % P == p))
    return keep.astype(F32)


def _exb(b):
    u = lax.broadcasted_iota(jnp.int32, (PH, L), 0)
    l = lax.broadcasted_iota(jnp.int32, (PH, L), 1)
    keep = ((u // H == l // PH) & ((u % H) % P == (l // H) % P)
            & ((u % H) // P == b))
    return keep.astype(F32)


def _ln(z, w, b):
    mp = _fold_p()
    cnt = float(N * H)
    cs = jnp.sum(z, axis=1)
    m = _mm(cs[:, None, :], mp)[:, 0, :] / cnt
    mb = _mm(m[:, None, :], mp.T)[:, 0, :]
    z0 = z - mb[:, None, :]
    s2 = jnp.sum(z0 * z0, axis=1)
    v = _mm(s2[:, None, :], mp)[:, 0, :] / cnt
    r = lax.rsqrt(v + 1e-5)
    rb = _mm(r[:, None, :], mp.T)[:, 0, :]
    return z0 * rb[:, None, :] * w + b


def _relu(x):
    return jnp.maximum(x, 0.0)


def _pre_body(x8, cond_r, pos32, deg2, table, attn,
              p1W, p1b, p2W, p2b,
              w1r, b1r, kin2, b2r, kp1, bp1r, kp2, bp2r,
              wc1k, bc1r, wc2k, bc2r, gnw0, gnb0, time_s,
              zs_out, acc_out, dis_out, te_out, condt_out):
    r0 = table[pl.ds(time_s[0], 1), :]
    r1 = table[pl.ds(time_s[1], 1), :]
    te = jnp.concatenate([r0, r1], axis=0)
    te = jax.nn.silu(_mm(te[:, None, :], p1W[...])[:, 0, :] + p1b[...])
    te = jax.nn.silu(_mm(te[:, None, :], p2W[...])[:, 0, :] + p2b[...])
    te_out[...] = te

    a = jax.nn.softmax(attn[...], axis=0)
    sp = _sp32()

    xt = x8[...]
    xw = jnp.stack([_mm(xt, _exb(0)), _mm(xt, _exb(1))])
    h = _relu(xw * w1r[...] + b1r[...])
    h = _mm(h, kin2[...]) + b2r[...]
    pp = _relu(_mm(pos32[...], kp1[...]) + bp1r[...])
    pp = _mm(pp, kp2[...]) + bp2r[...]
    h = h + _mm(pp, sp)
    acc_out[...] = h * a[0:1, :, None]

    conds = cond_r[...]
    ct = []
    for b in range(B):
        ab = jnp.concatenate([conds[b * P + p] for p in range(P)], axis=-1)
        qs = [_mm(ab, _rq(q)) for q in range(P)]
        ct.append(jnp.stack(qs, axis=1).reshape(M, L))
    cond = jnp.stack(ct)
    condt_out[...] = cond

    c0 = _relu(_mm(cond, wc1k[...]) + bc1r[...])
    c0 = _mm(c0, wc2k[...]) + bc2r[...]
    zn = _ln(h + c0, _mm(gnw0[...], sp), _mm(gnb0[...], sp))

    dgv = deg2[...]
    dg = dgv[0] + dgv[1]
    dis = jnp.where(dg > 0, lax.rsqrt(jnp.where(dg > 0, dg, 1.0)), 0.0)
    dis_out[...] = dis
    zs_out[...] = zn * dis


def _mid_body(g_in, cond, acc_in, dis, te, attn,
              wt1, bt1, wt2, bt2, tnw, tnb, tp, tcbr,
              wc1k, bc1r, wc2k, bc2r, gnw, gnb,
              zs_out, acc_out, *, jidx):
    a = jax.nn.softmax(attn[...], axis=0)
    tev = te[...]
    tb = _mm(_relu(_mm(tev[:, None, :], wt1[...])[:, 0, :] + bt1[...])[:, None, :],
             wt2[...])[:, 0, :] + bt2[...]
    tbrow = jnp.concatenate([tb] * (L // H), axis=1)

    sp = _sp32()
    disv = dis[...]
    z = g_in[...] * disv + tbrow[:, None, :]
    z = _ln(z, _mm(tnw[...], sp), _mm(tnb[...], sp))
    h = _mm(z, tp[...]) + tcbr[...]
    acc_out[...] = acc_in[...] + h * a[jidx + 1:jidx + 2, :, None]

    c = _relu(_mm(cond[...], wc1k[...]) + bc1r[...])
    c = _mm(c, wc2k[...]) + bc2r[...]
    zn = _ln(h + c, _mm(gnw[...], sp), _mm(gnb[...], sp))
    zs_out[...] = zn * disv


def _fin_body(g_in, acc_in, dis, te, attn,
              wt1, bt1, wt2, bt2, tnw, tnb, tp, tcbr,
              kout1, bo1r, kout2, bo2,
              o_out):
    a = jax.nn.softmax(attn[...], axis=0)
    tev = te[...]
    tb = _mm(_relu(_mm(tev[:, None, :], wt1[...])[:, 0, :] + bt1[...])[:, None, :],
             wt2[...])[:, 0, :] + bt2[...]
    tbrow = jnp.concatenate([tb] * (L // H), axis=1)

    sp = _sp32()
    z = g_in[...] * dis[...] + tbrow[:, None, :]
    z = _ln(z, _mm(tnw[...], sp), _mm(tnb[...], sp))
    h = _mm(z, tp[...]) + tcbr[...]
    acc = acc_in[...] + h * a[NB:NB + 1, :, None]

    u = _relu(_mm(acc, kout1[...]) + bo1r[...])
    o_out[...] = _mm(u, kout2[...]) + bo2[...]


def _tc_call(body, n_in, out_shapes, smem_last=False):
    specs = [pl.BlockSpec(memory_space=pltpu.VMEM) for _ in range(n_in)]
    if smem_last:
        specs[-1] = pl.BlockSpec(memory_space=pltpu.SMEM)
    return pl.pallas_call(
        body,
        in_specs=specs,
        out_specs=[pl.BlockSpec(memory_space=pltpu.VMEM)
                   for _ in range(len(out_shapes))],
        out_shape=out_shapes,
    )


def _tbl():
    steps = jnp.arange(500, dtype=F32)[:, None]
    dims = jnp.arange(16, dtype=F32)[None, :]
    e = steps * 10.0 ** (dims * 4.0 / 16.0)
    return jnp.concatenate([jnp.sin(e), jnp.cos(e)], axis=1)


def _kron(a, b):
    return jnp.einsum("ij,kl->ikjl", a, b).reshape(
        a.shape[0] * b.shape[0], a.shape[1] * b.shape[1])


def kernel(x, time, cond, pos, edge_index, p1W, p1b, p2W, p2b,
           inW1, inb1, inW2, inb2, posW1, posb1, posW2, posb2, attn,
           Wt1, bt1, Wt2, bt2, Wc1, bc1, Wc2, bc2, gnw, gnb, tnw, tnb,
           tcW, tcb, outW1, outb1, outW2, outb2):
    row = edge_index[0].astype(jnp.int32).reshape(TILES, CHUNKS, K)
    col = edge_index[1].astype(jnp.int32).reshape(TILES, CHUNKS, K)

    col_d = edge_index[1].astype(jnp.int32).reshape(CORES, TILES, DCH, K)
    deg2 = _get_deg()(col_d).reshape(CORES, M, L)

    i8 = jnp.eye(H, dtype=F32)
    i16 = jnp.eye(16, dtype=F32)
    i4 = jnp.eye(4, dtype=F32)

    def _rowv(v):
        return jnp.tile(v, L // v.shape[0])[None]

    kin2 = _kron(i16, inW2)
    kp1 = _kron(i4, posW1)
    kp2 = _kron(i4, posW2)
    kout1 = _kron(i16, outW1)
    kout2 = _kron(i16, outW2)
    w1r = _rowv(inW1[0])
    b1r = _rowv(inb1)
    b2r = _rowv(inb2)
    bp1r = jnp.tile(posb1, P)[None]
    bp2r = jnp.tile(posb2, P)[None]
    bo1r = _rowv(outb1)
    bo2 = jnp.broadcast_to(outb2[None], (1, 16))

    x8 = x.reshape(B * P, M, 4).transpose(1, 2, 0).reshape(M, PH)
    cond_r = cond.reshape(B * P, N // 16, 16 * H)
    pos32 = pos.reshape(M, PH)
    table = _tbl()

    def _blk(i):
        return dict(
            wc1k=_kron(i16, Wc1[i]), bc1r=_rowv(bc1[i]),
            wc2k=_kron(i16, Wc2[i]), bc2r=_rowv(bc2[i]),
            gnw=gnw[i].reshape(M, PH),
            gnb=gnb[i].reshape(M, PH),
            tnw=tnw[i].reshape(M, PH),
            tnb=tnb[i].reshape(M, PH),
            tp=_kron(i4, _kron(tcW[i].T, i8)),
            tcbr=jnp.tile(jnp.repeat(tcb[i], H), P)[None],
            wt1=Wt1[i], bt1=bt1[i][None], wt2=Wt2[i], bt2=bt2[i][None],
        )

    big = jax.ShapeDtypeStruct((B, M, L), F32)
    b0 = _blk(0)
    pre = _tc_call(_pre_body, 25, [
        big, big, jax.ShapeDtypeStruct((M, L), F32),
        jax.ShapeDtypeStruct((B, 64), F32), big], smem_last=True)
    zs, acc, dis, te, cond_t = pre(
        x8, cond_r, pos32, deg2, table, attn,
        p1W, p1b[None], p2W, p2b[None],
        w1r, b1r, kin2, b2r, kp1, bp1r, kp2, bp2r,
        b0["wc1k"], b0["bc1r"], b0["wc2k"], b0["bc2r"],
        b0["gnw"], b0["gnb"], time.astype(jnp.int32))

    for j in range(NB - 1):
        g = _lgconv(zs.reshape(CORES, N, PH), row, col).reshape(B, M, L)
        bj = _blk(j)
        bn = _blk(j + 1)
        mid = _tc_call(functools.partial(_mid_body, jidx=j), 20, [big, big])
        zs, acc = mid(
            g, cond_t, acc, dis, te, attn,
            bj["wt1"], bj["bt1"], bj["wt2"], bj["bt2"],
            bj["tnw"], bj["tnb"], bj["tp"], bj["tcbr"],
            bn["wc1k"], bn["bc1r"], bn["wc2k"], bn["bc2r"],
            bn["gnw"], bn["gnb"])

    g = _lgconv(zs.reshape(CORES, N, PH), row, col).reshape(B, M, L)
    b7 = _blk(NB - 1)
    fin = _tc_call(_fin_body, 17, [jax.ShapeDtypeStruct((B, M, 16), F32)])
    o2, = fin(
        g, acc, dis, te, attn,
        b7["wt1"], b7["bt1"], b7["wt2"], b7["bt2"],
        b7["tnw"], b7["tnb"], b7["tp"], b7["tcbr"],
        kout1, bo1r, kout2, bo2)

    return o2.reshape(B, N, P).transpose(0, 2, 1)
